# Initial kernel scaffold; baseline (speedup 1.0000x reference)
import jax
import jax.numpy as jnp
from jax.experimental import pallas as pl


def kernel(node_emb, er_src, er_dst, ee_src, ee_dst, ee_weight, rr_src, rr_dst, W_attn_w, W_attn_b, w0_w, w0_b, W1_w, W1_b, W2_w, W2_b, W3_w, W3_b):
    raise NotImplementedError("write your pallas kernel here")


# speedup vs baseline: 1.5793x; 1.5793x over previous
import functools

import jax
import jax.numpy as jnp
from jax.experimental import pallas as pl
from jax.experimental.pallas import tpu as pltpu

N_BLK = 1000


def _prologue_body(emb_ref, w_ref, b_ref, y_ref):
    y_ref[...] = (
        jnp.dot(emb_ref[...], w_ref[...], preferred_element_type=jnp.float32)
        + b_ref[...]
    )


def _dense_prologue(emb, wcat, bcat):
    n = emb.shape[0]
    return pl.pallas_call(
        _prologue_body,
        grid=(n // N_BLK,),
        in_specs=[
            pl.BlockSpec((N_BLK, 128), lambda i: (i, 0)),
            pl.BlockSpec((128, 640), lambda i: (0, 0)),
            pl.BlockSpec((1, 640), lambda i: (0, 0)),
        ],
        out_specs=pl.BlockSpec((N_BLK, 640), lambda i: (i, 0)),
        out_shape=jax.ShapeDtypeStruct((n, 640), jnp.float32),
    )(emb, wcat, bcat)


def _epilogue_body(x1_ref, a2_ref, ssum_ref, a3_ref, cnt_ref, b2_ref, b3_ref, o_ref):
    attn = a2_ref[...] / (ssum_ref[...] + 1e-9)
    mean = a3_ref[...] / jnp.maximum(cnt_ref[...], 1.0)
    o_ref[...] = (
        jnp.tanh(x1_ref[...])
        + jnp.tanh(attn + b2_ref[...])
        + jnp.tanh(mean + b3_ref[...])
    )


def _dense_epilogue(x1b, a2, ssum, a3, cnt, b2, b3):
    n = x1b.shape[0]
    return pl.pallas_call(
        _epilogue_body,
        grid=(n // N_BLK,),
        in_specs=[
            pl.BlockSpec((N_BLK, 128), lambda i: (i, 0)),
            pl.BlockSpec((N_BLK, 128), lambda i: (i, 0)),
            pl.BlockSpec((N_BLK, 1), lambda i: (i, 0)),
            pl.BlockSpec((N_BLK, 128), lambda i: (i, 0)),
            pl.BlockSpec((N_BLK, 1), lambda i: (i, 0)),
            pl.BlockSpec((1, 128), lambda i: (0, 0)),
            pl.BlockSpec((1, 128), lambda i: (0, 0)),
        ],
        out_specs=pl.BlockSpec((N_BLK, 128), lambda i: (i, 0)),
        out_shape=jax.ShapeDtypeStruct((n, 128), jnp.float32),
    )(x1b, a2, ssum, a3, cnt, b2, b3)


def kernel(node_emb, er_src, er_dst, ee_src, ee_dst, ee_weight, rr_src, rr_dst,
           W_attn_w, W_attn_b, w0_w, w0_b, W1_w, W1_b, W2_w, W2_b, W3_w, W3_b):
    n, d = node_emb.shape
    wcat = jnp.concatenate(
        [W_attn_w[:, :d].T, W_attn_w[:, d:].T, W1_w.T, W2_w.T, W3_w.T], axis=1)
    bcat = jnp.concatenate(
        [W_attn_b, jnp.zeros((d,), jnp.float32), W1_b,
         jnp.zeros((2 * d,), jnp.float32)])[None, :]
    y = _dense_prologue(node_emb, wcat, bcat)
    r_tab = y[:, 0:128]
    h_tab = y[:, 128:256]
    x1b = y[:, 256:384]
    v2 = y[:, 384:512]
    v3 = y[:, 512:640]

    s = jnp.tanh(r_tab[er_dst] + h_tab[er_src]) @ w0_w[0] + w0_b[0]
    ev = jnp.exp(s)
    ssum = jnp.zeros((n,), jnp.float32).at[er_src].add(ev)
    a2 = jnp.zeros((n, d), jnp.float32).at[er_src].add(ev[:, None] * v2[er_dst])
    a3 = (jnp.zeros((n, d), jnp.float32)
          .at[er_dst].add(v3[er_src])
          .at[ee_src].add(ee_weight[:, None] * v3[ee_dst])
          .at[rr_src].add(v3[rr_dst]))
    cnt = (jnp.zeros((n,), jnp.float32)
           .at[er_dst].add(1.0)
           .at[ee_src].add(ee_weight)
           .at[rr_src].add(1.0))

    return _dense_epilogue(x1b, a2, ssum[:, None], a3, cnt[:, None],
                           W2_b[None, :], W3_b[None, :])


# speedup vs baseline: 1.9551x; 1.2379x over previous
import functools

import jax
import jax.numpy as jnp
from jax import lax
from jax.experimental import pallas as pl
from jax.experimental.pallas import tpu as pltpu
from jax.experimental.pallas import tpu_sc as plsc

N_BLK = 1000
N = 10000
NPAD = 10112
DA = 144
NW = 32
B = 64

E_A = 320000
E_B1 = 480000
E_B2 = 160000
EPW_A, NB_A = 10048, 157
EPW_B1, NB_B1 = 15040, 235
EPW_B2, NB_B2 = 5056, 79


def _prologue_body(emb_ref, w_ref, b_ref, y_ref):
    y_ref[...] = (
        jnp.dot(emb_ref[...], w_ref[...], preferred_element_type=jnp.float32)
        + b_ref[...]
    )


def _dense_prologue(emb, wcat, bcat):
    n = emb.shape[0]
    return pl.pallas_call(
        _prologue_body,
        grid=(n // N_BLK,),
        in_specs=[
            pl.BlockSpec((N_BLK, 128), lambda i: (i, 0)),
            pl.BlockSpec((128, 640), lambda i: (0, 0)),
            pl.BlockSpec((1, 640), lambda i: (0, 0)),
        ],
        out_specs=pl.BlockSpec((N_BLK, 640), lambda i: (i, 0)),
        out_shape=jax.ShapeDtypeStruct((n, 640), jnp.float32),
    )(emb, wcat, bcat)


def _sc_body(r2, h2, va2, v3a, w0m2, znd, easrc, eadst, g1, s1, g2, s2, wee,
             acc2_out, acc3_out,
             gi_v, si_v, w0_v, wee_v, rbuf, hbuf, vbuf, acc, sem):
    c = lax.axis_index("c")
    s = lax.axis_index("s")
    wid = s * 2 + c
    rowbase = pl.multiple_of(s * 632, 8)
    iota16 = lax.iota(jnp.int32, 16)

    def zero_acc():
        pltpu.sync_copy(znd.at[pl.ds(rowbase, 632)], acc.at[pl.ds(rowbase, 632)])

    zero_acc()
    pltpu.sync_copy(w0m2, w0_v)
    plsc.subcore_barrier()

    ebase_a = wid * EPW_A

    def batch_a(b, _):
        off = pl.multiple_of(ebase_a + b * B, B)
        pltpu.sync_copy(easrc.at[pl.ds(off, B)], si_v)
        pltpu.sync_copy(eadst.at[pl.ds(off, B)], gi_v)
        cp1 = pltpu.async_copy(r2.at[gi_v], rbuf, sem)
        cp2 = pltpu.async_copy(h2.at[si_v], hbuf, sem)
        cp3 = pltpu.async_copy(va2.at[gi_v], vbuf, sem)
        cp1.wait()
        cp2.wait()
        cp3.wait()

        def group(g, _):
            rows = pl.multiple_of(g * 16, 16) + iota16
            acc_v = jnp.zeros((16,), jnp.float32)
            for dc in range(8):
                w0c = w0_v[pl.ds(dc * 16, 16)]
                for j in range(16):
                    cold = jnp.full((16,), dc * 16 + j, jnp.int32)
                    rv = plsc.load_gather(rbuf, [rows, cold])
                    hv = plsc.load_gather(hbuf, [rows, cold])
                    u = jnp.exp(rv + hv)
                    acc_v = acc_v + w0c[j] / (u + 1.0)
            ev = jnp.exp(acc_v)
            for dcol in range(DA):
                cold = jnp.full((16,), dcol, jnp.int32)
                v = plsc.load_gather(vbuf, [rows, cold])
                plsc.store_scatter(vbuf, [rows, cold], v * ev)
            return 0

        lax.fori_loop(0, B // 16, group, 0)
        pltpu.sync_copy(vbuf, acc.at[si_v], add=True)
        return 0

    lax.fori_loop(0, NB_A, batch_a, 0)

    plsc.subcore_barrier()
    pltpu.sync_copy(acc.at[pl.ds(rowbase, 632)],
                    acc2_out.at[c, pl.ds(rowbase, 632)])
    plsc.subcore_barrier()
    zero_acc()
    plsc.subcore_barrier()

    ebase1 = wid * EPW_B1

    def batch_b1(b, _):
        off = pl.multiple_of(ebase1 + b * B, B)
        pltpu.sync_copy(g1.at[pl.ds(off, B)], gi_v)
        pltpu.sync_copy(s1.at[pl.ds(off, B)], si_v)
        pltpu.async_copy(v3a.at[gi_v], vbuf, sem).wait()
        pltpu.sync_copy(vbuf, acc.at[si_v], add=True)
        return 0

    lax.fori_loop(0, NB_B1, batch_b1, 0)

    ebase2 = wid * EPW_B2

    def batch_b2(b, _):
        off = pl.multiple_of(ebase2 + b * B, B)
        pltpu.sync_copy(g2.at[pl.ds(off, B)], gi_v)
        pltpu.sync_copy(s2.at[pl.ds(off, B)], si_v)
        pltpu.sync_copy(wee.at[pl.ds(off, B)], wee_v)
        pltpu.async_copy(v3a.at[gi_v], vbuf, sem).wait()

        def group2(g, _):
            base = pl.multiple_of(g * 16, 16)
            rows = base + iota16
            wv = wee_v[pl.ds(base, 16)]
            for dcol in range(DA):
                cold = jnp.full((16,), dcol, jnp.int32)
                v = plsc.load_gather(vbuf, [rows, cold])
                plsc.store_scatter(vbuf, [rows, cold], v * wv)
            return 0

        lax.fori_loop(0, B // 16, group2, 0)
        pltpu.sync_copy(vbuf, acc.at[si_v], add=True)
        return 0

    lax.fori_loop(0, NB_B2, batch_b2, 0)

    plsc.subcore_barrier()
    pltpu.sync_copy(acc.at[pl.ds(rowbase, 632)],
                    acc3_out.at[c, pl.ds(rowbase, 632)])


_sc_aggregate = pl.kernel(
    _sc_body,
    out_type=(
        jax.ShapeDtypeStruct((2, NPAD, DA), jnp.float32),
        jax.ShapeDtypeStruct((2, NPAD, DA), jnp.float32),
    ),
    mesh=plsc.VectorSubcoreMesh(core_axis_name="c", subcore_axis_name="s"),
    compiler_params=pltpu.CompilerParams(use_tc_tiling_on_sc=False, needs_layout_passes=False),
    scratch_types=[
        pltpu.VMEM((B,), jnp.int32),
        pltpu.VMEM((B,), jnp.int32),
        pltpu.VMEM((128,), jnp.float32),
        pltpu.VMEM((B,), jnp.float32),
        pltpu.VMEM((B, 128), jnp.float32),
        pltpu.VMEM((B, 128), jnp.float32),
        pltpu.VMEM((B, DA), jnp.float32),
        pltpu.VMEM_SHARED((NPAD, DA), jnp.float32),
        pltpu.SemaphoreType.DMA,
    ],
)


def _epilogue_body(x1_ref, a2a_ref, a2b_ref, a3a_ref, a3b_ref, ec_ref,
                   b2_ref, b3_ref, o_ref):
    a2f = a2a_ref[...] + a2b_ref[...]
    a3f = a3a_ref[...] + a3b_ref[...]
    ec = ec_ref[...]
    ssum = a2f[:, 128:129] * ec + 1e-9
    cnt = jnp.maximum(a3f[:, 128:129], 1.0)
    o_ref[...] = (
        jnp.tanh(x1_ref[...])
        + jnp.tanh(a2f[:, :128] * ec / ssum + b2_ref[...])
        + jnp.tanh(a3f[:, :128] / cnt + b3_ref[...])
    )


def _dense_epilogue(x1b, a2a, a2b, a3a, a3b, ec, b2, b3):
    n = x1b.shape[0]
    acc_spec = pl.BlockSpec((N_BLK, DA), lambda i: (i, 0))
    return pl.pallas_call(
        _epilogue_body,
        grid=(n // N_BLK,),
        in_specs=[
            pl.BlockSpec((N_BLK, 128), lambda i: (i, 0)),
            acc_spec, acc_spec, acc_spec, acc_spec,
            pl.BlockSpec((1, 1), lambda i: (0, 0)),
            pl.BlockSpec((1, 128), lambda i: (0, 0)),
            pl.BlockSpec((1, 128), lambda i: (0, 0)),
        ],
        out_specs=pl.BlockSpec((N_BLK, 128), lambda i: (i, 0)),
        out_shape=jax.ShapeDtypeStruct((n, 128), jnp.float32),
    )(x1b, a2a, a2b, a3a, a3b, ec, b2, b3)


def _pad1(x, total, fill):
    return jnp.concatenate(
        [x, jnp.full((total - x.shape[0],), fill, x.dtype)])


def kernel(node_emb, er_src, er_dst, ee_src, ee_dst, ee_weight, rr_src, rr_dst,
           W_attn_w, W_attn_b, w0_w, w0_b, W1_w, W1_b, W2_w, W2_b, W3_w, W3_b):
    n, d = node_emb.shape
    f32 = jnp.float32
    wcat = jnp.concatenate(
        [2.0 * W_attn_w[:, :d].T, 2.0 * W_attn_w[:, d:].T,
         W1_w.T, W2_w.T, W3_w.T], axis=1)
    bcat = jnp.concatenate(
        [2.0 * W_attn_b, jnp.zeros((d,), f32), W1_b,
         jnp.zeros((2 * d,), f32)])[None, :]
    y = _dense_prologue(node_emb, wcat, bcat)

    rowpad = jnp.zeros((NPAD - n, 128), f32)
    r2 = jnp.concatenate([y[:, 0:128], rowpad], axis=0)
    h2 = jnp.concatenate([y[:, 128:256], rowpad], axis=0)
    x1b = y[:, 256:384]
    aug = jnp.concatenate(
        [jnp.ones((n, 1), f32), jnp.zeros((n, DA - 129), f32)], axis=1)
    rowpad_a = jnp.zeros((NPAD - n, DA), f32)
    va2 = jnp.concatenate(
        [jnp.concatenate([y[:, 384:512], aug], axis=1), rowpad_a], axis=0)
    v3a = jnp.concatenate(
        [jnp.concatenate([y[:, 512:640], aug], axis=1), rowpad_a], axis=0)

    w0m2 = -2.0 * w0_w[0]
    ec = jnp.exp(jnp.sum(w0_w) + w0_b[0]).reshape(1, 1)
    znd = jnp.zeros((NPAD, DA), f32)

    easrc = _pad1(er_src, EPW_A * NW, n)
    eadst = _pad1(er_dst, EPW_A * NW, n)
    g1 = _pad1(jnp.concatenate([er_src, rr_dst]), EPW_B1 * NW, n)
    s1 = _pad1(jnp.concatenate([er_dst, rr_src]), EPW_B1 * NW, n)
    g2 = _pad1(ee_dst, EPW_B2 * NW, n)
    s2 = _pad1(ee_src, EPW_B2 * NW, n)
    wee = _pad1(ee_weight, EPW_B2 * NW, 0.0)

    acc2, acc3 = _sc_aggregate(r2, h2, va2, v3a, w0m2, znd,
                               easrc, eadst, g1, s1, g2, s2, wee)

    return _dense_epilogue(x1b, acc2[0, :n], acc2[1, :n],
                           acc3[0, :n], acc3[1, :n], ec,
                           W2_b[None, :], W3_b[None, :])


# speedup vs baseline: 2.7724x; 1.4181x over previous
import jax
import jax.numpy as jnp
from jax import lax
from jax.experimental import pallas as pl
from jax.experimental.pallas import tpu as pltpu
from jax.experimental.pallas import tpu_sc as plsc

N_BLK = 1000
N = 10000
NPAD = 10112
DA = 144
NW = 32
B = 32

EPW_A, NB_A = 10048, 314
EPW_B1, NB_B1 = 15360, 480
EPW_B2, NB_B2 = 5184, 162


def _prologue_body(emb_ref, w_ref, b_ref, y_ref):
    y_ref[...] = (
        jnp.dot(emb_ref[...], w_ref[...], preferred_element_type=jnp.float32)
        + b_ref[...]
    )


def _dense_prologue(emb, wcat, bcat):
    n = emb.shape[0]
    return pl.pallas_call(
        _prologue_body,
        grid=(n // N_BLK,),
        in_specs=[
            pl.BlockSpec((N_BLK, 128), lambda i: (i, 0)),
            pl.BlockSpec((128, 640), lambda i: (0, 0)),
            pl.BlockSpec((1, 640), lambda i: (0, 0)),
        ],
        out_specs=pl.BlockSpec((N_BLK, 640), lambda i: (i, 0)),
        out_shape=jax.ShapeDtypeStruct((n, 640), jnp.float32),
    )(emb, wcat, bcat)


def _sc_body(r2, h2, va2, v3a, w0m2, znd, easrc, eadst, gsrc1, sdst1,
             gsrc2, sdst2, wee, acc2_out, acc3_out, *scr):
    bufs = scr[0:6]
    gis = scr[6:12]
    sis = scr[12:18]
    wees = scr[18:24]
    w0_v = scr[24]
    acc = scr[25]
    sem_i = scr[26:32]
    sem_g = scr[32:38]

    c = lax.axis_index("c")
    s = lax.axis_index("s")
    wid = s * 2 + c
    rowbase = pl.multiple_of(s * 632, 8)
    iota16 = lax.iota(jnp.int32, 16)

    def zero_acc():
        pltpu.sync_copy(znd.at[pl.ds(rowbase, 632)], acc.at[pl.ds(rowbase, 632)])

    zero_acc()
    pltpu.sync_copy(w0m2, w0_v)
    plsc.subcore_barrier()

    def fire_idx(k, b, garr, sarr, ebase, with_wee):
        off = pl.multiple_of(ebase + b * B, 8)
        pltpu.async_copy(garr.at[pl.ds(off, B)], gis[k], sem_i[k])
        pltpu.async_copy(sarr.at[pl.ds(off, B)], sis[k], sem_i[k])
        if with_wee:
            pltpu.async_copy(wee.at[pl.ds(off, B)], wees[k], sem_i[k])

    def drain_idx(k, with_wee):
        pltpu.make_async_copy(easrc.at[pl.ds(0, B)], gis[k], sem_i[k]).wait()
        pltpu.make_async_copy(easrc.at[pl.ds(0, B)], sis[k], sem_i[k]).wait()
        if with_wee:
            pltpu.make_async_copy(wee.at[pl.ds(0, B)], wees[k], sem_i[k]).wait()

    ebase_a = wid * EPW_A

    def fire_gathers_a(j):
        k = 3 * j
        drain_idx(k, False)
        pltpu.async_copy(r2.at[gis[k]], bufs[k], sem_g[k])
        pltpu.async_copy(h2.at[sis[k]], bufs[k + 1], sem_g[k])
        pltpu.async_copy(va2.at[gis[k]], bufs[k + 2], sem_g[k])

    def wait_gathers_a(j):
        k = 3 * j
        pltpu.make_async_copy(r2.at[gis[k]], bufs[k], sem_g[k]).wait()
        pltpu.make_async_copy(h2.at[sis[k]], bufs[k + 1], sem_g[k]).wait()
        pltpu.make_async_copy(va2.at[gis[k]], bufs[k + 2], sem_g[k]).wait()

    def compute_a(j):
        k = 3 * j
        rb, hb, vb = bufs[k], bufs[k + 1], bufs[k + 2]

        def group(g, _):
            rows = pl.multiple_of(g * 16, 16) + iota16

            def dchunk(dc, acc_v):
                w0c = w0_v[pl.ds(pl.multiple_of(dc * 16, 16), 16)]
                base = dc * 16
                for jj in range(16):
                    cold = jnp.full((16,), jj, jnp.int32) + base
                    rv = plsc.load_gather(rb, [rows, cold])
                    hv = plsc.load_gather(hb, [rows, cold])
                    u = jnp.exp(rv + hv)
                    acc_v = acc_v + w0c[jj] / (u + 1.0)
                return acc_v

            ev = jnp.exp(lax.fori_loop(0, 8, dchunk,
                                       jnp.zeros((16,), jnp.float32)))

            def vchunk(vc, _):
                base = vc * 16
                for jj in range(16):
                    cold = jnp.full((16,), jj, jnp.int32) + base
                    v = plsc.load_gather(vb, [rows, cold])
                    plsc.store_scatter(vb, [rows, cold], v * ev)
                return 0

            lax.fori_loop(0, DA // 16, vchunk, 0)
            return 0

        lax.fori_loop(0, B // 16, group, 0)
        pltpu.sync_copy(vb, acc.at[sis[k]], add=True)

    fire_idx(0, 0, eadst, easrc, ebase_a, False)
    fire_idx(3, 1, eadst, easrc, ebase_a, False)
    fire_gathers_a(0)
    fire_gathers_a(1)

    def pair_a(p, _):
        b0 = p * 2
        wait_gathers_a(0)
        compute_a(0)
        fire_idx(0, b0 + 2, eadst, easrc, ebase_a, False)
        wait_gathers_a(1)
        compute_a(1)
        fire_idx(3, b0 + 3, eadst, easrc, ebase_a, False)
        fire_gathers_a(0)
        fire_gathers_a(1)
        return 0

    lax.fori_loop(0, NB_A // 2 - 1, pair_a, 0)
    wait_gathers_a(0)
    compute_a(0)
    wait_gathers_a(1)
    compute_a(1)

    plsc.subcore_barrier()
    pltpu.sync_copy(acc.at[pl.ds(rowbase, 632)],
                    acc2_out.at[c, pl.ds(rowbase, 632)])
    plsc.subcore_barrier()
    zero_acc()
    plsc.subcore_barrier()

    def fire_gather_b(k):
        pltpu.async_copy(v3a.at[gis[k]], bufs[k], sem_g[k])

    def wait_gather_b(k):
        pltpu.make_async_copy(v3a.at[gis[k]], bufs[k], sem_g[k]).wait()

    def scatter_b(k):
        pltpu.sync_copy(bufs[k], acc.at[sis[k]], add=True)

    def scale_b2(k):
        def group2(g, _):
            base = pl.multiple_of(g * 16, 16)
            rows = base + iota16
            wv = wees[k][pl.ds(base, 16)]

            def vchunk(vc, _):
                cbase = vc * 16
                for jj in range(16):
                    cold = jnp.full((16,), jj, jnp.int32) + cbase
                    v = plsc.load_gather(bufs[k], [rows, cold])
                    plsc.store_scatter(bufs[k], [rows, cold], v * wv)
                return 0

            lax.fori_loop(0, DA // 16, vchunk, 0)
            return 0

        lax.fori_loop(0, B // 16, group2, 0)

    ebase1 = wid * EPW_B1
    for k in range(6):
        fire_idx(k, k, gsrc1, sdst1, ebase1, False)
    for k in range(6):
        drain_idx(k, False)
        fire_gather_b(k)

    def six_b1(q, _):
        b = q * 6
        for k in range(6):
            wait_gather_b(k)
            scatter_b(k)
            fire_idx(k, b + 6 + k, gsrc1, sdst1, ebase1, False)
        for k in range(6):
            drain_idx(k, False)
            fire_gather_b(k)
        return 0

    lax.fori_loop(0, NB_B1 // 6 - 1, six_b1, 0)
    for k in range(6):
        wait_gather_b(k)
        scatter_b(k)

    ebase2 = wid * EPW_B2
    for k in range(6):
        fire_idx(k, k, gsrc2, sdst2, ebase2, True)
    for k in range(6):
        drain_idx(k, True)
        fire_gather_b(k)

    def six_b2(q, _):
        b = q * 6
        for k in range(6):
            wait_gather_b(k)
            scale_b2(k)
            scatter_b(k)
            fire_idx(k, b + 6 + k, gsrc2, sdst2, ebase2, True)
        for k in range(6):
            drain_idx(k, True)
            fire_gather_b(k)
        return 0

    lax.fori_loop(0, NB_B2 // 6 - 1, six_b2, 0)
    for k in range(6):
        wait_gather_b(k)
        scale_b2(k)
        scatter_b(k)

    plsc.subcore_barrier()
    pltpu.sync_copy(acc.at[pl.ds(rowbase, 632)],
                    acc3_out.at[c, pl.ds(rowbase, 632)])


_sc_aggregate = pl.kernel(
    _sc_body,
    out_type=(
        jax.ShapeDtypeStruct((2, NPAD, DA), jnp.float32),
        jax.ShapeDtypeStruct((2, NPAD, DA), jnp.float32),
    ),
    mesh=plsc.VectorSubcoreMesh(core_axis_name="c", subcore_axis_name="s"),
    compiler_params=pltpu.CompilerParams(
        use_tc_tiling_on_sc=False, needs_layout_passes=False),
    scratch_types=(
        [pltpu.VMEM((B, DA), jnp.float32)] * 6
        + [pltpu.VMEM((B,), jnp.int32)] * 12
        + [pltpu.VMEM((B,), jnp.float32)] * 6
        + [pltpu.VMEM((128,), jnp.float32)]
        + [pltpu.VMEM_SHARED((NPAD, DA), jnp.float32)]
        + [pltpu.SemaphoreType.DMA] * 12
    ),
)


def _epilogue_body(x1_ref, a2a_ref, a2b_ref, a3a_ref, a3b_ref, ec_ref,
                   b2_ref, b3_ref, o_ref):
    a2f = a2a_ref[...] + a2b_ref[...]
    a3f = a3a_ref[...] + a3b_ref[...]
    ec = ec_ref[...]
    ssum = a2f[:, 128:129] * ec + 1e-9
    cnt = jnp.maximum(a3f[:, 128:129], 1.0)
    o_ref[...] = (
        jnp.tanh(x1_ref[...])
        + jnp.tanh(a2f[:, :128] * ec / ssum + b2_ref[...])
        + jnp.tanh(a3f[:, :128] / cnt + b3_ref[...])
    )


def _dense_epilogue(x1b, a2a, a2b, a3a, a3b, ec, b2, b3):
    n = x1b.shape[0]
    acc_spec = pl.BlockSpec((N_BLK, DA), lambda i: (i, 0))
    return pl.pallas_call(
        _epilogue_body,
        grid=(n // N_BLK,),
        in_specs=[
            pl.BlockSpec((N_BLK, 128), lambda i: (i, 0)),
            acc_spec, acc_spec, acc_spec, acc_spec,
            pl.BlockSpec((1, 1), lambda i: (0, 0)),
            pl.BlockSpec((1, 128), lambda i: (0, 0)),
            pl.BlockSpec((1, 128), lambda i: (0, 0)),
        ],
        out_specs=pl.BlockSpec((N_BLK, 128), lambda i: (i, 0)),
        out_shape=jax.ShapeDtypeStruct((n, 128), jnp.float32),
    )(x1b, a2a, a2b, a3a, a3b, ec, b2, b3)


def _pad1(x, total, fill):
    return jnp.concatenate(
        [x, jnp.full((total - x.shape[0],), fill, x.dtype)])


def kernel(node_emb, er_src, er_dst, ee_src, ee_dst, ee_weight, rr_src, rr_dst,
           W_attn_w, W_attn_b, w0_w, w0_b, W1_w, W1_b, W2_w, W2_b, W3_w, W3_b):
    n, d = node_emb.shape
    f32 = jnp.float32
    wcat = jnp.concatenate(
        [2.0 * W_attn_w[:, :d].T, 2.0 * W_attn_w[:, d:].T,
         W1_w.T, W2_w.T, W3_w.T], axis=1)
    bcat = jnp.concatenate(
        [2.0 * W_attn_b, jnp.zeros((d,), f32), W1_b,
         jnp.zeros((2 * d,), f32)])[None, :]
    y = _dense_prologue(node_emb, wcat, bcat)

    colpad = jnp.zeros((n, DA - 128), f32)
    rowpad_a = jnp.zeros((NPAD - n, DA), f32)
    r2 = jnp.concatenate(
        [jnp.concatenate([y[:, 0:128], colpad], axis=1), rowpad_a], axis=0)
    h2 = jnp.concatenate(
        [jnp.concatenate([y[:, 128:256], colpad], axis=1), rowpad_a], axis=0)
    x1b = y[:, 256:384]
    aug = jnp.concatenate(
        [jnp.ones((n, 1), f32), jnp.zeros((n, DA - 129), f32)], axis=1)
    va2 = jnp.concatenate(
        [jnp.concatenate([y[:, 384:512], aug], axis=1), rowpad_a], axis=0)
    v3a = jnp.concatenate(
        [jnp.concatenate([y[:, 512:640], aug], axis=1), rowpad_a], axis=0)

    w0m2 = -2.0 * w0_w[0]
    ec = jnp.exp(jnp.sum(w0_w) + w0_b[0]).reshape(1, 1)
    znd = jnp.zeros((NPAD, DA), f32)

    easrc = _pad1(er_src, EPW_A * NW, n)
    eadst = _pad1(er_dst, EPW_A * NW, n)
    gsrc1 = _pad1(jnp.concatenate([er_src, rr_dst]), EPW_B1 * NW, n)
    sdst1 = _pad1(jnp.concatenate([er_dst, rr_src]), EPW_B1 * NW, n)
    gsrc2 = _pad1(ee_dst, EPW_B2 * NW, n)
    sdst2 = _pad1(ee_src, EPW_B2 * NW, n)
    wee = _pad1(ee_weight, EPW_B2 * NW, 0.0)

    acc2, acc3 = _sc_aggregate(r2, h2, va2, v3a, w0m2, znd,
                               easrc, eadst, gsrc1, sdst1, gsrc2, sdst2, wee)

    return _dense_epilogue(x1b, acc2[0, :n], acc2[1, :n],
                           acc3[0, :n], acc3[1, :n], ec,
                           W2_b[None, :], W3_b[None, :])


# speedup vs baseline: 3.7287x; 1.3449x over previous
import jax
import jax.numpy as jnp
from jax import lax
from jax.experimental import pallas as pl
from jax.experimental.pallas import tpu as pltpu
from jax.experimental.pallas import tpu_sc as plsc

N_BLK = 1000
N = 10000
NPAD = 10112
DA = 144
NW = 32
B = 32

EPW_A, NB_A = 10048, 314
EPW_B1, NB_B1 = 15360, 480
EPW_B2, NB_B2 = 5184, 162


def _prologue_body(emb_ref, w_ref, b_ref, y_ref):
    y_ref[...] = (
        jnp.dot(emb_ref[...], w_ref[...], preferred_element_type=jnp.float32)
        + b_ref[...]
    )


def _dense_prologue(emb, wcat, bcat):
    n = emb.shape[0]
    return pl.pallas_call(
        _prologue_body,
        grid=(n // N_BLK,),
        in_specs=[
            pl.BlockSpec((N_BLK, 128), lambda i: (i, 0)),
            pl.BlockSpec((128, 640), lambda i: (0, 0)),
            pl.BlockSpec((1, 640), lambda i: (0, 0)),
        ],
        out_specs=pl.BlockSpec((N_BLK, 640), lambda i: (i, 0)),
        out_shape=jax.ShapeDtypeStruct((n, 640), jnp.float32),
    )(emb, wcat, bcat)


def _sc_body(r2, h2, va2, v3a, w0m2, znd, easrc, eadst, gsrc1, sdst1,
             gsrc2, sdst2, wee, acc2_out, acc3_out, *scr):
    bufs = scr[0:6]
    gis = scr[6:12]
    sis = scr[12:18]
    wees = scr[18:24]
    w0_v = scr[24]
    acc = scr[25]
    sem_i = scr[26:32]
    sem_g = scr[32:38]

    c = lax.axis_index("c")
    s = lax.axis_index("s")
    wid = s * 2 + c
    rowbase = pl.multiple_of(s * 632, 8)
    iota16 = lax.iota(jnp.int32, 16)

    def zero_acc():
        pltpu.sync_copy(znd.at[pl.ds(rowbase, 632)], acc.at[pl.ds(rowbase, 632)])

    zero_acc()
    pltpu.sync_copy(w0m2, w0_v)
    plsc.subcore_barrier()

    def fire_idx(k, b, garr, sarr, ebase, with_wee):
        off = pl.multiple_of(ebase + b * B, 8)
        pltpu.async_copy(garr.at[pl.ds(off, B)], gis[k], sem_i[k])
        pltpu.async_copy(sarr.at[pl.ds(off, B)], sis[k], sem_i[k])
        if with_wee:
            pltpu.async_copy(wee.at[pl.ds(off, B)], wees[k], sem_i[k])

    def drain_idx(k, with_wee):
        pltpu.make_async_copy(easrc.at[pl.ds(0, B)], gis[k], sem_i[k]).wait()
        pltpu.make_async_copy(easrc.at[pl.ds(0, B)], sis[k], sem_i[k]).wait()
        if with_wee:
            pltpu.make_async_copy(wee.at[pl.ds(0, B)], wees[k], sem_i[k]).wait()

    ebase_a = wid * EPW_A

    def fire_gathers_a(j):
        k = 3 * j
        drain_idx(k, False)
        pltpu.async_copy(r2.at[gis[k]], bufs[k], sem_g[k])
        pltpu.async_copy(h2.at[sis[k]], bufs[k + 1], sem_g[k])
        pltpu.async_copy(va2.at[gis[k]], bufs[k + 2], sem_g[k])

    def wait_gathers_a(j):
        k = 3 * j
        pltpu.make_async_copy(r2.at[gis[k]], bufs[k], sem_g[k]).wait()
        pltpu.make_async_copy(h2.at[sis[k]], bufs[k + 1], sem_g[k]).wait()
        pltpu.make_async_copy(va2.at[gis[k]], bufs[k + 2], sem_g[k]).wait()

    def compute_a(j):
        k = 3 * j
        rb, hb, vb = bufs[k], bufs[k + 1], bufs[k + 2]

        def group(g, _):
            rows = pl.multiple_of(g * 16, 16) + iota16

            def dchunk(dc, acc_v):
                base = dc * 16
                for jj in range(16):
                    cold = base + ((iota16 + jj) & 15)
                    rv = plsc.load_gather(rb, [rows, cold])
                    hv = plsc.load_gather(hb, [rows, cold])
                    w0v = plsc.load_gather(w0_v, [cold])
                    u = jnp.exp(rv + hv)
                    acc_v = acc_v + w0v / (u + 1.0)
                return acc_v

            ev = jnp.exp(lax.fori_loop(0, 8, dchunk,
                                       jnp.zeros((16,), jnp.float32)))

            def vchunk(vc, _):
                base = vc * 16
                for jj in range(16):
                    cold = base + ((iota16 + jj) & 15)
                    v = plsc.load_gather(vb, [rows, cold])
                    plsc.store_scatter(vb, [rows, cold], v * ev)
                return 0

            lax.fori_loop(0, DA // 16, vchunk, 0)
            return 0

        lax.fori_loop(0, B // 16, group, 0)
        pltpu.sync_copy(vb, acc.at[sis[k]], add=True)

    fire_idx(0, 0, eadst, easrc, ebase_a, False)
    fire_idx(3, 1, eadst, easrc, ebase_a, False)
    fire_gathers_a(0)
    fire_gathers_a(1)

    def pair_a(p, _):
        b0 = p * 2
        wait_gathers_a(0)
        compute_a(0)
        fire_idx(0, b0 + 2, eadst, easrc, ebase_a, False)
        wait_gathers_a(1)
        compute_a(1)
        fire_idx(3, b0 + 3, eadst, easrc, ebase_a, False)
        fire_gathers_a(0)
        fire_gathers_a(1)
        return 0

    lax.fori_loop(0, NB_A // 2 - 1, pair_a, 0)
    wait_gathers_a(0)
    compute_a(0)
    wait_gathers_a(1)
    compute_a(1)

    plsc.subcore_barrier()
    pltpu.sync_copy(acc.at[pl.ds(rowbase, 632)],
                    acc2_out.at[c, pl.ds(rowbase, 632)])
    plsc.subcore_barrier()
    zero_acc()
    plsc.subcore_barrier()

    def fire_gather_b(k):
        pltpu.async_copy(v3a.at[gis[k]], bufs[k], sem_g[k])

    def wait_gather_b(k):
        pltpu.make_async_copy(v3a.at[gis[k]], bufs[k], sem_g[k]).wait()

    def scatter_b(k):
        pltpu.sync_copy(bufs[k], acc.at[sis[k]], add=True)

    def scale_b2(k):
        def group2(g, _):
            base = pl.multiple_of(g * 16, 16)
            rows = base + iota16
            wv = wees[k][pl.ds(base, 16)]

            def vchunk(vc, _):
                cbase = vc * 16
                for jj in range(16):
                    cold = cbase + ((iota16 + jj) & 15)
                    v = plsc.load_gather(bufs[k], [rows, cold])
                    plsc.store_scatter(bufs[k], [rows, cold], v * wv)
                return 0

            lax.fori_loop(0, DA // 16, vchunk, 0)
            return 0

        lax.fori_loop(0, B // 16, group2, 0)

    ebase1 = wid * EPW_B1
    for k in range(6):
        fire_idx(k, k, gsrc1, sdst1, ebase1, False)
    for k in range(6):
        drain_idx(k, False)
        fire_gather_b(k)

    def six_b1(q, _):
        b = q * 6
        for k in range(6):
            wait_gather_b(k)
            scatter_b(k)
            fire_idx(k, b + 6 + k, gsrc1, sdst1, ebase1, False)
        for k in range(6):
            drain_idx(k, False)
            fire_gather_b(k)
        return 0

    lax.fori_loop(0, NB_B1 // 6 - 1, six_b1, 0)
    for k in range(6):
        wait_gather_b(k)
        scatter_b(k)

    ebase2 = wid * EPW_B2
    for k in range(6):
        fire_idx(k, k, gsrc2, sdst2, ebase2, True)
    for k in range(6):
        drain_idx(k, True)
        fire_gather_b(k)

    def six_b2(q, _):
        b = q * 6
        for k in range(6):
            wait_gather_b(k)
            scale_b2(k)
            scatter_b(k)
            fire_idx(k, b + 6 + k, gsrc2, sdst2, ebase2, True)
        for k in range(6):
            drain_idx(k, True)
            fire_gather_b(k)
        return 0

    lax.fori_loop(0, NB_B2 // 6 - 1, six_b2, 0)
    for k in range(6):
        wait_gather_b(k)
        scale_b2(k)
        scatter_b(k)

    plsc.subcore_barrier()
    pltpu.sync_copy(acc.at[pl.ds(rowbase, 632)],
                    acc3_out.at[c, pl.ds(rowbase, 632)])


_sc_aggregate = pl.kernel(
    _sc_body,
    out_type=(
        jax.ShapeDtypeStruct((2, NPAD, DA), jnp.float32),
        jax.ShapeDtypeStruct((2, NPAD, DA), jnp.float32),
    ),
    mesh=plsc.VectorSubcoreMesh(core_axis_name="c", subcore_axis_name="s"),
    compiler_params=pltpu.CompilerParams(
        use_tc_tiling_on_sc=False, needs_layout_passes=False),
    scratch_types=(
        [pltpu.VMEM((B, DA), jnp.float32)] * 6
        + [pltpu.VMEM((B,), jnp.int32)] * 12
        + [pltpu.VMEM((B,), jnp.float32)] * 6
        + [pltpu.VMEM((128,), jnp.float32)]
        + [pltpu.VMEM_SHARED((NPAD, DA), jnp.float32)]
        + [pltpu.SemaphoreType.DMA] * 12
    ),
)


def _epilogue_body(x1_ref, a2a_ref, a2b_ref, a3a_ref, a3b_ref, ec_ref,
                   b2_ref, b3_ref, o_ref):
    a2f = a2a_ref[...] + a2b_ref[...]
    a3f = a3a_ref[...] + a3b_ref[...]
    ec = ec_ref[...]
    ssum = a2f[:, 128:129] * ec + 1e-9
    cnt = jnp.maximum(a3f[:, 128:129], 1.0)
    o_ref[...] = (
        jnp.tanh(x1_ref[...])
        + jnp.tanh(a2f[:, :128] * ec / ssum + b2_ref[...])
        + jnp.tanh(a3f[:, :128] / cnt + b3_ref[...])
    )


def _dense_epilogue(x1b, a2a, a2b, a3a, a3b, ec, b2, b3):
    n = x1b.shape[0]
    acc_spec = pl.BlockSpec((N_BLK, DA), lambda i: (i, 0))
    return pl.pallas_call(
        _epilogue_body,
        grid=(n // N_BLK,),
        in_specs=[
            pl.BlockSpec((N_BLK, 128), lambda i: (i, 0)),
            acc_spec, acc_spec, acc_spec, acc_spec,
            pl.BlockSpec((1, 1), lambda i: (0, 0)),
            pl.BlockSpec((1, 128), lambda i: (0, 0)),
            pl.BlockSpec((1, 128), lambda i: (0, 0)),
        ],
        out_specs=pl.BlockSpec((N_BLK, 128), lambda i: (i, 0)),
        out_shape=jax.ShapeDtypeStruct((n, 128), jnp.float32),
    )(x1b, a2a, a2b, a3a, a3b, ec, b2, b3)


def _pad1(x, total, fill):
    return jnp.concatenate(
        [x, jnp.full((total - x.shape[0],), fill, x.dtype)])


def kernel(node_emb, er_src, er_dst, ee_src, ee_dst, ee_weight, rr_src, rr_dst,
           W_attn_w, W_attn_b, w0_w, w0_b, W1_w, W1_b, W2_w, W2_b, W3_w, W3_b):
    n, d = node_emb.shape
    f32 = jnp.float32
    wcat = jnp.concatenate(
        [2.0 * W_attn_w[:, :d].T, 2.0 * W_attn_w[:, d:].T,
         W1_w.T, W2_w.T, W3_w.T], axis=1)
    bcat = jnp.concatenate(
        [2.0 * W_attn_b, jnp.zeros((d,), f32), W1_b,
         jnp.zeros((2 * d,), f32)])[None, :]
    y = _dense_prologue(node_emb, wcat, bcat)

    colpad = jnp.zeros((n, DA - 128), f32)
    rowpad_a = jnp.zeros((NPAD - n, DA), f32)
    r2 = jnp.concatenate(
        [jnp.concatenate([y[:, 0:128], colpad], axis=1), rowpad_a], axis=0)
    h2 = jnp.concatenate(
        [jnp.concatenate([y[:, 128:256], colpad], axis=1), rowpad_a], axis=0)
    x1b = y[:, 256:384]
    aug = jnp.concatenate(
        [jnp.ones((n, 1), f32), jnp.zeros((n, DA - 129), f32)], axis=1)
    va2 = jnp.concatenate(
        [jnp.concatenate([y[:, 384:512], aug], axis=1), rowpad_a], axis=0)
    v3a = jnp.concatenate(
        [jnp.concatenate([y[:, 512:640], aug], axis=1), rowpad_a], axis=0)

    w0m2 = -2.0 * w0_w[0]
    ec = jnp.exp(jnp.sum(w0_w) + w0_b[0]).reshape(1, 1)
    znd = jnp.zeros((NPAD, DA), f32)

    easrc = _pad1(er_src, EPW_A * NW, n)
    eadst = _pad1(er_dst, EPW_A * NW, n)
    gsrc1 = _pad1(jnp.concatenate([er_src, rr_dst]), EPW_B1 * NW, n)
    sdst1 = _pad1(jnp.concatenate([er_dst, rr_src]), EPW_B1 * NW, n)
    gsrc2 = _pad1(ee_dst, EPW_B2 * NW, n)
    sdst2 = _pad1(ee_src, EPW_B2 * NW, n)
    wee = _pad1(ee_weight, EPW_B2 * NW, 0.0)

    acc2, acc3 = _sc_aggregate(r2, h2, va2, v3a, w0m2, znd,
                               easrc, eadst, gsrc1, sdst1, gsrc2, sdst2, wee)

    return _dense_epilogue(x1b, acc2[0, :n], acc2[1, :n],
                           acc3[0, :n], acc3[1, :n], ec,
                           W2_b[None, :], W3_b[None, :])
